# baseline (speedup 1.0000x reference)
import jax
import jax.numpy as jnp
import numpy as np
from jax.experimental import pallas as pl
from jax.experimental.pallas import tpu as pltpu
from jax.sharding import Mesh, PartitionSpec as P


def _se_body(x_ref, w1_ref, b1_ref, w2_ref, b2_ref, o_ref):
    x = x_ref[...]
    pooled = jnp.mean(x, axis=2)
    h = jnp.dot(pooled, w1_ref[...], preferred_element_type=jnp.float32)
    h = jnp.maximum(h + b1_ref[...], 0.0)
    g = jnp.dot(h, w2_ref[...], preferred_element_type=jnp.float32)
    g = jax.nn.sigmoid(g + b2_ref[...])
    o_ref[...] = (x * g[:, :, None].astype(x.dtype)).astype(o_ref.dtype)


def _se_shard(x, w1, b1, w2, b2):
    Bs, C, H, W = x.shape
    HW = H * W
    hidden = w1.shape[1]

    x3 = x.reshape(Bs, C, HW)
    bt = 16
    while Bs % bt:
        bt //= 2
    grid = (Bs // bt,)

    out = pl.pallas_call(
        _se_body,
        out_shape=jax.ShapeDtypeStruct((Bs, C, HW), x.dtype),
        grid=grid,
        in_specs=[
            pl.BlockSpec((bt, C, HW), lambda b: (b, 0, 0)),
            pl.BlockSpec((C, hidden), lambda b: (0, 0)),
            pl.BlockSpec((1, hidden), lambda b: (0, 0)),
            pl.BlockSpec((hidden, C), lambda b: (0, 0)),
            pl.BlockSpec((1, C), lambda b: (0, 0)),
        ],
        out_specs=pl.BlockSpec((bt, C, HW), lambda b: (b, 0, 0)),
        compiler_params=pltpu.CompilerParams(
            dimension_semantics=("arbitrary",),
            vmem_limit_bytes=56 * 1024 * 1024,
        ),
        cost_estimate=pl.CostEstimate(
            flops=3 * Bs * C * HW + 4 * Bs * C * hidden,
            transcendentals=Bs * C,
            bytes_accessed=2 * Bs * C * HW * 4,
        ),
    )(x3, w1, b1.reshape(1, hidden), w2, b2.reshape(1, C))

    return out.reshape(Bs, C, H, W)


def kernel(x, w1, b1, w2, b2):
    devs = jax.devices()
    B = x.shape[0]
    if len(devs) >= 2 and B % 2 == 0:
        mesh = Mesh(np.array(devs[:2]), ("b",))
        fn = jax.shard_map(
            _se_shard, mesh=mesh,
            in_specs=(P("b"), P(), P(), P(), P()),
            out_specs=P("b"),
            check_vma=False,
        )
        return fn(x, w1, b1, w2, b2)
    return _se_shard(x, w1, b1, w2, b2)


# speedup vs baseline: 4.4951x; 4.4951x over previous
import jax
import jax.numpy as jnp
from jax.experimental import pallas as pl
from jax.experimental.pallas import tpu as pltpu


def _se_body(x_ref, w1_ref, b1_ref, w2_ref, b2_ref, o_ref):
    x = x_ref[...]
    pooled = jnp.mean(x, axis=1)
    h = jnp.dot(pooled, w1_ref[...], preferred_element_type=jnp.float32)
    h = jnp.maximum(h + b1_ref[...], 0.0)
    g = jnp.dot(h, w2_ref[...], preferred_element_type=jnp.float32)
    g = jax.nn.sigmoid(g + b2_ref[...])
    o_ref[...] = (x * g[:, None, :].astype(x.dtype)).astype(o_ref.dtype)


def kernel(x, w1, b1, w2, b2):
    B, C, H, W = x.shape
    HW = H * W
    hidden = w1.shape[1]
    itemsize = jnp.dtype(x.dtype).itemsize

    xt = x.reshape(B, C, HW).transpose(0, 2, 1)

    lanes = -(-C // 128) * 128
    sub = -(-HW // 8) * 8
    win = sub * lanes * itemsize
    max_bt = (50 * 1024 * 1024) // (4 * win)
    bt = int(max(1, min(max_bt, pl.cdiv(B, 2))))
    grid = (int(pl.cdiv(B, bt)),)

    block = (bt, HW, C)
    out = pl.pallas_call(
        _se_body,
        out_shape=jax.ShapeDtypeStruct((B, HW, C), x.dtype),
        grid=grid,
        in_specs=[
            pl.BlockSpec(block, lambda b: (b, 0, 0)),
            pl.BlockSpec((C, hidden), lambda b: (0, 0)),
            pl.BlockSpec((1, hidden), lambda b: (0, 0)),
            pl.BlockSpec((hidden, C), lambda b: (0, 0)),
            pl.BlockSpec((1, C), lambda b: (0, 0)),
        ],
        out_specs=pl.BlockSpec(block, lambda b: (b, 0, 0)),
        compiler_params=pltpu.CompilerParams(
            dimension_semantics=("parallel",),
            vmem_limit_bytes=60 * 1024 * 1024,
        ),
        cost_estimate=pl.CostEstimate(
            flops=3 * B * C * HW + 4 * B * C * hidden,
            transcendentals=B * C,
            bytes_accessed=2 * B * C * HW * itemsize,
        ),
    )(xt, w1, b1.reshape(1, hidden), w2, b2.reshape(1, C))

    return out.transpose(0, 2, 1).reshape(B, C, H, W)
